# baseline (speedup 1.0000x reference)
import functools

import jax
import jax.numpy as jnp
from jax import lax
from jax.experimental import pallas as pl
from jax.experimental.pallas import tpu as pltpu
from jax.experimental.pallas import tpu_sc as plsc

N_EMBD = 1024
NUM_CORES = 2
NUM_SUBCORES = 16
NUM_WORKERS = NUM_CORES * NUM_SUBCORES
CHUNK = 16
NBUF = 4


DEPTH = 2


def _emb_body(table_hbm, idx_hbm, out_hbm, idx_v, bufs, *sems):
    nchunk = idx_hbm.shape[1]
    gsems, ssems = sems[:NBUF], sems[NBUF:]
    wid = lax.axis_index("s") * NUM_CORES + lax.axis_index("c")
    base = wid * (nchunk * CHUNK)

    def out_slice(cur):
        return out_hbm.at[pl.ds(base + cur * CHUNK, CHUNK)]

    idx_rows = idx_v

    def gather(j, bj):
        pltpu.async_copy(table_hbm.at[idx_rows.at[j]], bufs.at[bj], gsems[bj])

    pltpu.sync_copy(idx_hbm.at[wid], idx_v)

    for j in range(DEPTH):
        gather(j, j % NBUF)

    def outer(g):
        for b in range(NBUF):
            cur = g * NBUF + b
            j = cur + DEPTH
            bj = (b + DEPTH) % NBUF

            @pl.when(j < nchunk)
            def _():
                @pl.when(j >= NBUF)
                def _():
                    pltpu.make_async_copy(
                        bufs.at[bj], out_slice(j - NBUF), ssems[bj]
                    ).wait()

                gather(j, bj)

            pltpu.make_async_copy(
                table_hbm.at[idx_rows.at[cur]], bufs.at[b], gsems[b]
            ).wait()
            pltpu.async_copy(bufs.at[b], out_slice(cur), ssems[b])

    pl.loop(0, nchunk // NBUF)(outer)

    for i in range(NBUF):
        c = nchunk - NBUF + i
        pltpu.make_async_copy(bufs.at[c % NBUF], out_slice(c), ssems[c % NBUF]).wait()


def _copy_body(src_ref, dst_ref):
    dst_ref[...] = src_ref[...]


def _tc_copy(x, blk=1024):
    rows, cols = x.shape
    return pl.pallas_call(
        _copy_body,
        grid=(rows // blk,),
        in_specs=[pl.BlockSpec((blk, cols), lambda i: (i, 0))],
        out_specs=pl.BlockSpec((blk, cols), lambda i: (i, 0)),
        out_shape=jax.ShapeDtypeStruct((rows, cols), x.dtype),
    )(x)


def _sc_embedding_lookup(emb_table, idx):
    n_ids = idx.size
    nchunk = idx.shape[1]
    grid_kernel = pl.kernel(
        _emb_body,
        out_type=jax.ShapeDtypeStruct((n_ids, N_EMBD), jnp.float32),
        mesh=plsc.VectorSubcoreMesh(
            core_axis_name="c",
            subcore_axis_name="s",
            num_cores=NUM_CORES,
            num_subcores=NUM_SUBCORES,
        ),
        scratch_types=[
            pltpu.VMEM((nchunk, CHUNK), jnp.int32),
            pltpu.VMEM((NBUF, CHUNK, N_EMBD), jnp.float32),
        ] + [pltpu.SemaphoreType.DMA] * (2 * NBUF),
    )
    return grid_kernel(emb_table, idx)


def kernel(token_ids, encoder_context, emb_table):
    batch, seq_len = token_ids.shape
    n_ids = batch * seq_len
    nchunk = n_ids // (NUM_WORKERS * CHUNK)
    idx = token_ids.astype(jnp.int32).reshape(NUM_WORKERS, nchunk, CHUNK)
    flat = _sc_embedding_lookup(emb_table, idx)
    ctx = _tc_copy(encoder_context.reshape(n_ids, N_EMBD))
    return (flat.reshape(batch, seq_len, N_EMBD), ctx.reshape(batch, seq_len, N_EMBD))


# speedup vs baseline: 1.0011x; 1.0011x over previous
import jax
import jax.numpy as jnp
from jax import lax
from jax.experimental import pallas as pl
from jax.experimental.pallas import tpu as pltpu
from jax.experimental.pallas import tpu_sc as plsc

N_EMBD = 1024
NUM_CORES = 2
NUM_SUBCORES = 16
NUM_WORKERS = NUM_CORES * NUM_SUBCORES
CHUNK = 16
NBUF = 4


DEPTH = 2


def _emb_body(table_hbm, idx_hbm, out_hbm, idx_v, bufs, *sems):
    nchunk = idx_hbm.shape[1]
    gsems, ssems = sems[:NBUF], sems[NBUF:]
    wid = lax.axis_index("s") * NUM_CORES + lax.axis_index("c")
    base = wid * (nchunk * CHUNK)

    def out_slice(cur):
        return out_hbm.at[pl.ds(base + cur * CHUNK, CHUNK)]

    idx_rows = idx_v

    def gather(j, bj):
        pltpu.async_copy(table_hbm.at[idx_rows.at[j]], bufs.at[bj], gsems[bj])

    pltpu.sync_copy(idx_hbm.at[wid], idx_v)

    for j in range(DEPTH):
        gather(j, j % NBUF)

    def outer(g):
        for b in range(NBUF):
            cur = g * NBUF + b
            j = cur + DEPTH
            bj = (b + DEPTH) % NBUF

            @pl.when(j < nchunk)
            def _():
                @pl.when(j >= NBUF)
                def _():
                    pltpu.make_async_copy(
                        bufs.at[bj], out_slice(j - NBUF), ssems[bj]
                    ).wait()

                gather(j, bj)

            pltpu.make_async_copy(
                table_hbm.at[idx_rows.at[cur]], bufs.at[b], gsems[b]
            ).wait()
            pltpu.async_copy(bufs.at[b], out_slice(cur), ssems[b])

    pl.loop(0, nchunk // NBUF)(outer)

    for i in range(NBUF):
        c = nchunk - NBUF + i
        pltpu.make_async_copy(bufs.at[c % NBUF], out_slice(c), ssems[c % NBUF]).wait()


def _copy_body(src_ref, dst_ref):
    dst_ref[...] = src_ref[...]


def _tc_copy(x, blk=1024):
    rows, cols = x.shape
    return pl.pallas_call(
        _copy_body,
        grid=(rows // blk,),
        in_specs=[pl.BlockSpec((blk, cols), lambda i: (i, 0))],
        out_specs=pl.BlockSpec((blk, cols), lambda i: (i, 0)),
        out_shape=jax.ShapeDtypeStruct((rows, cols), x.dtype),
    )(x)


def _sc_embedding_lookup(emb_table, idx):
    n_ids = idx.size
    nchunk = idx.shape[1]
    grid_kernel = pl.kernel(
        _emb_body,
        out_type=jax.ShapeDtypeStruct((n_ids, N_EMBD), jnp.float32),
        mesh=plsc.VectorSubcoreMesh(
            core_axis_name="c",
            subcore_axis_name="s",
            num_cores=NUM_CORES,
            num_subcores=NUM_SUBCORES,
        ),
        scratch_types=[
            pltpu.VMEM((nchunk, CHUNK), jnp.int32),
            pltpu.VMEM((NBUF, CHUNK, N_EMBD), jnp.float32),
        ] + [pltpu.SemaphoreType.DMA] * (2 * NBUF),
    )
    return grid_kernel(emb_table, idx)


def kernel(token_ids, encoder_context, emb_table):
    batch, seq_len = token_ids.shape
    n_ids = batch * seq_len
    nchunk = n_ids // (NUM_WORKERS * CHUNK)
    idx = token_ids.astype(jnp.int32).reshape(NUM_WORKERS, nchunk, CHUNK)
    flat = _sc_embedding_lookup(emb_table, idx)
    ctx = _tc_copy(encoder_context.reshape(n_ids, N_EMBD))
    return (flat.reshape(batch, seq_len, N_EMBD), ctx.reshape(batch, seq_len, N_EMBD))
